# baseline (speedup 1.0000x reference)
import functools

import jax
import jax.numpy as jnp
from jax import lax
from jax.experimental import pallas as pl
from jax.experimental.pallas import tpu as pltpu
from jax.experimental.pallas import tpu_sc as plsc

D_IN = 1024
NL = 32768
K = 64
NT = 2048

LT = 1024
NSTEP = NL // LT
BPB = 128
NBS = LT // BPB
CAP = 128

NC, NS, LANES = 2, 16, 16
NW = NC * NS
TPW = NT // NW


TT = 1024
NTT = NT // TT


def _encode_body(x_ref, w_ref, benc_ref, bdec_ref, fa_ref, bm_ref, sae_scr):
    i = pl.program_id(0)
    j = pl.program_id(1)

    @pl.when(jnp.logical_and(i == 0, j == 0))
    def _():
        sae_scr[...] = x_ref[...] - bdec_ref[...]

    lhs = sae_scr[pl.ds(j * TT, TT), :]
    pre = lax.dot_general(lhs, w_ref[...],
                          (((1,), (1,)), ((), ())),
                          preferred_element_type=jnp.float32)
    fa = jnp.maximum(pre + benc_ref[0], 0.0)
    fa_ref[...] = fa
    bm_ref[0] = jnp.max(fa.reshape(TT, NBS, BPB), axis=2).T


def _encode(x, w_enc, b_enc, b_dec):
    return pl.pallas_call(
        _encode_body,
        grid=(NSTEP, NTT),
        in_specs=[
            pl.BlockSpec((NT, D_IN), lambda i, j: (0, 0)),
            pl.BlockSpec((LT, D_IN), lambda i, j: (i, 0)),
            pl.BlockSpec((1, 1, LT), lambda i, j: (i, 0, 0)),
            pl.BlockSpec((1, D_IN), lambda i, j: (0, 0)),
        ],
        out_specs=[
            pl.BlockSpec((TT, LT), lambda i, j: (j, i)),
            pl.BlockSpec((1, NBS, TT), lambda i, j: (i, 0, j)),
        ],
        out_shape=[
            jax.ShapeDtypeStruct((NT, NL), jnp.float32),
            jax.ShapeDtypeStruct((NSTEP, NBS, NT), jnp.float32),
        ],
        scratch_shapes=[
            pltpu.VMEM((NT, D_IN), jnp.float32),
        ],
        compiler_params=pltpu.CompilerParams(
            vmem_limit_bytes=36 * 1024 * 1024),
    )(x, w_enc, b_enc.reshape(NSTEP, 1, LT), b_dec.reshape(1, D_IN))


def _tau_body(bm_ref, tau_ref):
    bits = lax.bitcast_convert_type(
        bm_ref[...].reshape(NSTEP * NBS, NT), jnp.int32)

    def search(it, t):
        cand = t | (jnp.int32(1) << (30 - it))
        cnt = jnp.sum((bits >= cand).astype(jnp.int32), axis=0,
                      keepdims=True)
        return jnp.where(cnt >= K, cand, t)

    t = lax.fori_loop(0, 31, search, jnp.zeros((1, NT), jnp.int32))
    tau_ref[...] = lax.bitcast_convert_type(t, jnp.float32)


def _tau(bm):
    return pl.pallas_call(
        _tau_body,
        out_shape=jax.ShapeDtypeStruct((1, NT), jnp.float32),
    )(bm)


def _sc_compact_body(fa_hbm, tau_hbm, cidx_hbm, cval_hbm,
                     tau_v, row_v, ci_v, cv_v):
    wid = lax.axis_index("s") * NC + lax.axis_index("c")
    base = wid * TPW
    pltpu.sync_copy(tau_hbm.at[pl.ds(base, TPW)], tau_v.at[pl.ds(0, TPW)])

    def tok_body(ti, _):
        token = base + ti
        pltpu.sync_copy(fa_hbm.at[token], row_v)
        for q in range(CAP // LANES):
            ci_v[pl.ds(q * LANES, LANES)] = jnp.zeros((LANES,), jnp.int32)
            cv_v[pl.ds(q * LANES, LANES)] = jnp.full((LANES,), -1.0,
                                                     jnp.float32)
        tau_vec = jnp.full((LANES,), tau_v[pl.ds(ti, LANES)][0])

        def vbody(j, off):
            v = row_v[pl.ds(j * LANES, LANES)]
            m = v >= tau_vec
            scan = plsc.cumsum(m.astype(jnp.int32))
            pos = off + scan - 1
            ok = jnp.logical_and(m, pos < CAP)
            idxv = lax.iota(jnp.int32, LANES) + j * LANES
            plsc.store_scatter(ci_v, [pos], idxv, mask=ok)
            plsc.store_scatter(cv_v, [pos], v, mask=ok)
            return off + plsc.all_reduce_population_count(m)

        lax.fori_loop(0, NL // LANES, vbody, jnp.zeros((LANES,), jnp.int32),
                      unroll=8)
        pltpu.sync_copy(ci_v, cidx_hbm.at[token])
        pltpu.sync_copy(cv_v, cval_hbm.at[token])
        return 0

    lax.fori_loop(0, TPW, tok_body, 0)


def _sc_compact(fa, tau):
    kern = pl.kernel(
        _sc_compact_body,
        out_type=[
            jax.ShapeDtypeStruct((NT, CAP), jnp.int32),
            jax.ShapeDtypeStruct((NT, CAP), jnp.float32),
        ],
        mesh=plsc.VectorSubcoreMesh(core_axis_name="c", subcore_axis_name="s"),
        compiler_params=pltpu.CompilerParams(needs_layout_passes=False),
        scratch_types=[
            pltpu.VMEM((TPW + LANES,), jnp.float32),
            pltpu.VMEM((NL,), jnp.float32),
            pltpu.VMEM((CAP,), jnp.int32),
            pltpu.VMEM((CAP,), jnp.float32),
        ],
    )
    return kern(fa, tau)


def _select_body(cv_ref, ci_ref, ta_ref, ti_ref, val_scr):
    val_scr[...] = cv_ref[...]
    idx = ci_ref[...]
    for k in range(K):
        val = val_scr[...]
        m = jnp.max(val, axis=1, keepdims=True)
        sel = jnp.min(jnp.where(val == m, idx, jnp.int32(2 ** 30)),
                      axis=1, keepdims=True)
        ta_ref[:, pl.ds(k, 1)] = m
        ti_ref[:, pl.ds(k, 1)] = sel
        val_scr[...] = jnp.where(idx == sel, -jnp.inf, val)


def _select(cval, cidx):
    return pl.pallas_call(
        _select_body,
        out_shape=[
            jax.ShapeDtypeStruct((NT, K), jnp.float32),
            jax.ShapeDtypeStruct((NT, K), jnp.int32),
        ],
        scratch_shapes=[pltpu.VMEM((NT, CAP), jnp.float32)],
    )(cval, cidx)


def _sc_decode_body(wdec_hbm, ti_hbm, ta_hbm, y_hbm,
                    idx_v, act_v, rows_v, y_v, sem):
    wid = lax.axis_index("s") * NC + lax.axis_index("c")
    base = wid * TPW

    def tok_body(t, _):
        token = base + t
        pltpu.sync_copy(ti_hbm.at[pl.ds(token * K, K)], idx_v)
        pltpu.sync_copy(ta_hbm.at[pl.ds(token * K, K)], act_v.at[pl.ds(0, K)])
        pltpu.async_copy(wdec_hbm.at[idx_v], rows_v, sem).wait()
        for chunk in range(D_IN // 256):

            def rbody(r, acc):
                a = jnp.full((LANES,), act_v[pl.ds(r, LANES)][0])
                return tuple(
                    acc[q] + rows_v[r, pl.ds(chunk * 256 + q * LANES, LANES)] * a
                    for q in range(16))

            acc = lax.fori_loop(
                0, K, rbody,
                tuple(jnp.zeros((LANES,), jnp.float32) for _ in range(16)),
                unroll=4)
            for q in range(16):
                y_v[pl.ds(chunk * 256 + q * LANES, LANES)] = acc[q]
        pltpu.sync_copy(y_v, y_hbm.at[pl.ds(token * D_IN, D_IN)])
        return 0

    lax.fori_loop(0, TPW, tok_body, 0)


def _sc_decode(w_dec, top_idx, top_acts):
    kern = pl.kernel(
        _sc_decode_body,
        out_type=jax.ShapeDtypeStruct((NT * D_IN,), jnp.float32),
        mesh=plsc.VectorSubcoreMesh(core_axis_name="c", subcore_axis_name="s"),
        compiler_params=pltpu.CompilerParams(needs_layout_passes=False),
        scratch_types=[
            pltpu.VMEM((K,), jnp.int32),
            pltpu.VMEM((K + LANES,), jnp.float32),
            pltpu.VMEM((K, D_IN), jnp.float32),
            pltpu.VMEM((D_IN,), jnp.float32),
            pltpu.SemaphoreType.DMA,
        ],
    )
    return kern(w_dec, top_idx.reshape(NT * K),
                top_acts.reshape(NT * K)).reshape(NT, D_IN)


def _final_body(x_ref, y_ref, bdec_ref, out_ref, fvu_ref, l2_ref):
    x = x_ref[...]
    so = y_ref[...] + bdec_ref[...]
    out_ref[...] = so
    e = so - x
    l2 = jnp.sum(e * e)
    mu = jnp.mean(x, axis=0, keepdims=True)
    tv = jnp.sum((x - mu) ** 2)
    l2_ref[...] = l2.reshape(1, 1)
    fvu_ref[...] = (l2 / tv).reshape(1, 1)


def _finalize(x, y, b_dec):
    return pl.pallas_call(
        _final_body,
        out_shape=[
            jax.ShapeDtypeStruct((NT, D_IN), jnp.float32),
            jax.ShapeDtypeStruct((1, 1), jnp.float32),
            jax.ShapeDtypeStruct((1, 1), jnp.float32),
        ],
    )(x, y, b_dec.reshape(1, D_IN))


def kernel(x, W_enc, b_enc, W_dec, b_dec):
    fa, bm = _encode(x, W_enc, b_enc, b_dec)
    tau = _tau(bm)
    cidx, cval = _sc_compact(fa, tau.reshape(NT))
    top_acts, top_idx = _select(cval, cidx)
    y = _sc_decode(W_dec, top_idx, top_acts)
    sae_out, fvu, l2 = _finalize(x, y, b_dec)
    z = jnp.zeros((), x.dtype)
    return (sae_out, fa, top_acts, top_idx, fvu.reshape(()), z, z, z,
            l2.reshape(()))


# speedup vs baseline: 2.3868x; 2.3868x over previous
import functools

import jax
import jax.numpy as jnp
from jax import lax
from jax.experimental import pallas as pl
from jax.experimental.pallas import tpu as pltpu
from jax.experimental.pallas import tpu_sc as plsc

D_IN = 1024
NL = 32768
K = 64
NT = 2048

LT = 1024
NSTEP = NL // LT
BPB = 128
NBS = LT // BPB
CAP = 128

NC, NS, LANES = 2, 16, 16
NW = NC * NS
TPW = NT // NW


TT = 1024
NTT = NT // TT


def _encode_body(x_ref, w_ref, benc_ref, bdec_ref, fa_ref, bm_ref, sae_scr):
    i = pl.program_id(0)
    j = pl.program_id(1)

    @pl.when(jnp.logical_and(i == 0, j == 0))
    def _():
        sae_scr[...] = x_ref[...] - bdec_ref[...]

    lhs = sae_scr[pl.ds(j * TT, TT), :]
    pre = lax.dot_general(lhs, w_ref[...],
                          (((1,), (1,)), ((), ())),
                          preferred_element_type=jnp.float32)
    fa = jnp.maximum(pre + benc_ref[0], 0.0)
    fa_ref[...] = fa
    bm_ref[0] = jnp.max(fa.reshape(TT, NBS, BPB), axis=2).T


def _encode(x, w_enc, b_enc, b_dec):
    return pl.pallas_call(
        _encode_body,
        grid=(NSTEP, NTT),
        in_specs=[
            pl.BlockSpec((NT, D_IN), lambda i, j: (0, 0)),
            pl.BlockSpec((LT, D_IN), lambda i, j: (i, 0)),
            pl.BlockSpec((1, 1, LT), lambda i, j: (i, 0, 0)),
            pl.BlockSpec((1, D_IN), lambda i, j: (0, 0)),
        ],
        out_specs=[
            pl.BlockSpec((TT, LT), lambda i, j: (j, i)),
            pl.BlockSpec((1, NBS, TT), lambda i, j: (i, 0, j)),
        ],
        out_shape=[
            jax.ShapeDtypeStruct((NT, NL), jnp.float32),
            jax.ShapeDtypeStruct((NSTEP, NBS, NT), jnp.float32),
        ],
        scratch_shapes=[
            pltpu.VMEM((NT, D_IN), jnp.float32),
        ],
        compiler_params=pltpu.CompilerParams(
            vmem_limit_bytes=36 * 1024 * 1024),
    )(x, w_enc, b_enc.reshape(NSTEP, 1, LT), b_dec.reshape(1, D_IN))


def _tau_body(bm_ref, tau_ref):
    bits = lax.bitcast_convert_type(
        bm_ref[...].reshape(NSTEP * NBS, NT), jnp.int32)

    def search(it, t):
        cand = t | (jnp.int32(1) << (30 - it))
        cnt = jnp.sum((bits >= cand).astype(jnp.int32), axis=0,
                      keepdims=True)
        return jnp.where(cnt >= K, cand, t)

    t = lax.fori_loop(0, 31, search, jnp.zeros((1, NT), jnp.int32))
    tau_ref[...] = lax.bitcast_convert_type(t, jnp.float32)


def _tau(bm):
    return pl.pallas_call(
        _tau_body,
        out_shape=jax.ShapeDtypeStruct((1, NT), jnp.float32),
    )(bm)


def _sc_compact_body(fa_hbm, tau_hbm, cidx_hbm, cval_hbm,
                     tau_v, row_v, ci_v, cv_v):
    wid = lax.axis_index("s") * NC + lax.axis_index("c")
    base = wid * TPW
    pltpu.sync_copy(tau_hbm.at[pl.ds(base, TPW)], tau_v.at[pl.ds(0, TPW)])

    def tok_body(ti, _):
        token = base + ti
        pltpu.sync_copy(fa_hbm.at[token], row_v)
        for q in range(CAP // LANES):
            ci_v[pl.ds(q * LANES, LANES)] = jnp.zeros((LANES,), jnp.int32)
            cv_v[pl.ds(q * LANES, LANES)] = jnp.full((LANES,), -1.0,
                                                     jnp.float32)
        tau_vec = jnp.full((LANES,), tau_v[pl.ds(ti, LANES)][0])

        @plsc.parallel_loop(0, NL // LANES,
                            carry=jnp.zeros((LANES,), jnp.int32), unroll=8)
        def vbody(j, off):
            v = row_v[pl.ds(j * LANES, LANES)]
            m = v >= tau_vec
            scan = plsc.cumsum(m.astype(jnp.int32))
            pos = off + scan - 1
            ok = jnp.logical_and(m, pos < CAP)
            idxv = lax.iota(jnp.int32, LANES) + j * LANES
            plsc.store_scatter(ci_v, [pos], idxv, mask=ok)
            plsc.store_scatter(cv_v, [pos], v, mask=ok)
            return off + plsc.all_reduce_population_count(m)
        pltpu.sync_copy(ci_v, cidx_hbm.at[token])
        pltpu.sync_copy(cv_v, cval_hbm.at[token])
        return 0

    lax.fori_loop(0, TPW, tok_body, 0)


def _sc_compact(fa, tau):
    kern = pl.kernel(
        _sc_compact_body,
        out_type=[
            jax.ShapeDtypeStruct((NT, CAP), jnp.int32),
            jax.ShapeDtypeStruct((NT, CAP), jnp.float32),
        ],
        mesh=plsc.VectorSubcoreMesh(core_axis_name="c", subcore_axis_name="s"),
        compiler_params=pltpu.CompilerParams(needs_layout_passes=False),
        scratch_types=[
            pltpu.VMEM((TPW + LANES,), jnp.float32),
            pltpu.VMEM((NL,), jnp.float32),
            pltpu.VMEM((CAP,), jnp.int32),
            pltpu.VMEM((CAP,), jnp.float32),
        ],
    )
    return kern(fa, tau)


def _select_body(cv_ref, ci_ref, ta_ref, ti_ref, val_scr):
    val_scr[...] = cv_ref[...]
    idx = ci_ref[...]
    for k in range(K):
        val = val_scr[...]
        m = jnp.max(val, axis=1, keepdims=True)
        sel = jnp.min(jnp.where(val == m, idx, jnp.int32(2 ** 30)),
                      axis=1, keepdims=True)
        ta_ref[:, pl.ds(k, 1)] = m
        ti_ref[:, pl.ds(k, 1)] = sel
        val_scr[...] = jnp.where(idx == sel, -jnp.inf, val)


def _select(cval, cidx):
    return pl.pallas_call(
        _select_body,
        out_shape=[
            jax.ShapeDtypeStruct((NT, K), jnp.float32),
            jax.ShapeDtypeStruct((NT, K), jnp.int32),
        ],
        scratch_shapes=[pltpu.VMEM((NT, CAP), jnp.float32)],
    )(cval, cidx)


def _sc_decode_body(wdec_hbm, ti_hbm, ta_hbm, y_hbm,
                    idx_v, act_v, rows_v, y_v, sem):
    wid = lax.axis_index("s") * NC + lax.axis_index("c")
    base = wid * TPW

    def tok_body(t, _):
        token = base + t
        pltpu.sync_copy(ti_hbm.at[pl.ds(token * K, K)], idx_v)
        pltpu.sync_copy(ta_hbm.at[pl.ds(token * K, K)], act_v.at[pl.ds(0, K)])
        pltpu.async_copy(wdec_hbm.at[idx_v], rows_v, sem).wait()
        for chunk in range(D_IN // 256):

            def rbody(r, acc):
                a = jnp.full((LANES,), act_v[pl.ds(r, LANES)][0])
                return tuple(
                    acc[q] + rows_v[r, pl.ds(chunk * 256 + q * LANES, LANES)] * a
                    for q in range(16))

            acc = lax.fori_loop(
                0, K, rbody,
                tuple(jnp.zeros((LANES,), jnp.float32) for _ in range(16)),
                unroll=4)
            for q in range(16):
                y_v[pl.ds(chunk * 256 + q * LANES, LANES)] = acc[q]
        pltpu.sync_copy(y_v, y_hbm.at[pl.ds(token * D_IN, D_IN)])
        return 0

    lax.fori_loop(0, TPW, tok_body, 0)


def _sc_decode(w_dec, top_idx, top_acts):
    kern = pl.kernel(
        _sc_decode_body,
        out_type=jax.ShapeDtypeStruct((NT * D_IN,), jnp.float32),
        mesh=plsc.VectorSubcoreMesh(core_axis_name="c", subcore_axis_name="s"),
        compiler_params=pltpu.CompilerParams(needs_layout_passes=False),
        scratch_types=[
            pltpu.VMEM((K,), jnp.int32),
            pltpu.VMEM((K + LANES,), jnp.float32),
            pltpu.VMEM((K, D_IN), jnp.float32),
            pltpu.VMEM((D_IN,), jnp.float32),
            pltpu.SemaphoreType.DMA,
        ],
    )
    return kern(w_dec, top_idx.reshape(NT * K),
                top_acts.reshape(NT * K)).reshape(NT, D_IN)


def _final_body(x_ref, y_ref, bdec_ref, out_ref, fvu_ref, l2_ref):
    x = x_ref[...]
    so = y_ref[...] + bdec_ref[...]
    out_ref[...] = so
    e = so - x
    l2 = jnp.sum(e * e)
    mu = jnp.mean(x, axis=0, keepdims=True)
    tv = jnp.sum((x - mu) ** 2)
    l2_ref[...] = l2.reshape(1, 1)
    fvu_ref[...] = (l2 / tv).reshape(1, 1)


def _finalize(x, y, b_dec):
    return pl.pallas_call(
        _final_body,
        out_shape=[
            jax.ShapeDtypeStruct((NT, D_IN), jnp.float32),
            jax.ShapeDtypeStruct((1, 1), jnp.float32),
            jax.ShapeDtypeStruct((1, 1), jnp.float32),
        ],
    )(x, y, b_dec.reshape(1, D_IN))


def kernel(x, W_enc, b_enc, W_dec, b_dec):
    fa, bm = _encode(x, W_enc, b_enc, b_dec)
    tau = _tau(bm)
    cidx, cval = _sc_compact(fa, tau.reshape(NT))
    top_acts, top_idx = _select(cval, cidx)
    y = _sc_decode(W_dec, top_idx, top_acts)
    sae_out, fvu, l2 = _finalize(x, y, b_dec)
    z = jnp.zeros((), x.dtype)
    return (sae_out, fa, top_acts, top_idx, fvu.reshape(()), z, z, z,
            l2.reshape(()))


# speedup vs baseline: 2.3890x; 1.0009x over previous
import functools

import jax
import jax.numpy as jnp
from jax import lax
from jax.experimental import pallas as pl
from jax.experimental.pallas import tpu as pltpu
from jax.experimental.pallas import tpu_sc as plsc

D_IN = 1024
NL = 32768
K = 64
NT = 2048

LT = 1024
NSTEP = NL // LT
BPB = 128
NBS = LT // BPB
CAP = 128

NC, NS, LANES = 2, 16, 16
NW = NC * NS
TPW = NT // NW


TT = 1024
NTT = NT // TT


def _encode_body(x_ref, w_ref, benc_ref, bdec_ref, fa_ref, bm_ref, sae_scr):
    i = pl.program_id(0)
    j = pl.program_id(1)

    @pl.when(jnp.logical_and(i == 0, j == 0))
    def _():
        sae_scr[...] = x_ref[...] - bdec_ref[...]

    lhs = sae_scr[pl.ds(j * TT, TT), :]
    pre = lax.dot_general(lhs, w_ref[...],
                          (((1,), (1,)), ((), ())),
                          preferred_element_type=jnp.float32)
    fa = jnp.maximum(pre + benc_ref[0], 0.0)
    fa_ref[...] = fa
    bm_ref[0] = jnp.max(fa.reshape(TT, NBS, BPB), axis=2).T


def _encode(x, w_enc, b_enc, b_dec):
    return pl.pallas_call(
        _encode_body,
        grid=(NSTEP, NTT),
        in_specs=[
            pl.BlockSpec((NT, D_IN), lambda i, j: (0, 0)),
            pl.BlockSpec((LT, D_IN), lambda i, j: (i, 0)),
            pl.BlockSpec((1, 1, LT), lambda i, j: (i, 0, 0)),
            pl.BlockSpec((1, D_IN), lambda i, j: (0, 0)),
        ],
        out_specs=[
            pl.BlockSpec((TT, LT), lambda i, j: (j, i)),
            pl.BlockSpec((1, NBS, TT), lambda i, j: (i, 0, j)),
        ],
        out_shape=[
            jax.ShapeDtypeStruct((NT, NL), jnp.float32),
            jax.ShapeDtypeStruct((NSTEP, NBS, NT), jnp.float32),
        ],
        scratch_shapes=[
            pltpu.VMEM((NT, D_IN), jnp.float32),
        ],
        compiler_params=pltpu.CompilerParams(
            vmem_limit_bytes=36 * 1024 * 1024),
    )(x, w_enc, b_enc.reshape(NSTEP, 1, LT), b_dec.reshape(1, D_IN))


def _tau_body(bm_ref, tau_ref):
    bits = lax.bitcast_convert_type(
        bm_ref[...].reshape(NSTEP * NBS, NT), jnp.int32)

    def search(it, t):
        cand = t | (jnp.int32(1) << (30 - it))
        cnt = jnp.sum((bits >= cand).astype(jnp.int32), axis=0,
                      keepdims=True)
        return jnp.where(cnt >= K, cand, t)

    t = lax.fori_loop(0, 31, search, jnp.zeros((1, NT), jnp.int32))
    tau_ref[...] = lax.bitcast_convert_type(t, jnp.float32)


def _tau(bm):
    return pl.pallas_call(
        _tau_body,
        out_shape=jax.ShapeDtypeStruct((1, NT), jnp.float32),
    )(bm)


def _sc_compact_body(fa_hbm, tau_hbm, cidx_hbm, cval_hbm,
                     tau_v, row_v, ci_v, cv_v):
    wid = lax.axis_index("s") * NC + lax.axis_index("c")
    base = wid * TPW
    pltpu.sync_copy(tau_hbm.at[pl.ds(base, TPW)], tau_v.at[pl.ds(0, TPW)])

    def tok_body(ti, _):
        token = base + ti
        pltpu.sync_copy(fa_hbm.at[token], row_v)
        for q in range(CAP // LANES):
            ci_v[pl.ds(q * LANES, LANES)] = jnp.zeros((LANES,), jnp.int32)
            cv_v[pl.ds(q * LANES, LANES)] = jnp.full((LANES,), -1.0,
                                                     jnp.float32)
        tau_vec = jnp.full((LANES,), tau_v[pl.ds(ti, LANES)][0])

        @plsc.parallel_loop(0, NL // LANES,
                            carry=jnp.zeros((LANES,), jnp.int32), unroll=8)
        def vbody(j, off):
            v = row_v[pl.ds(j * LANES, LANES)]
            m = v >= tau_vec
            scan = plsc.cumsum(m.astype(jnp.int32))
            pos = off + scan - 1
            ok = jnp.logical_and(m, pos < CAP)
            idxv = lax.iota(jnp.int32, LANES) + j * LANES
            plsc.store_scatter(ci_v, [pos], idxv, mask=ok)
            plsc.store_scatter(cv_v, [pos], v, mask=ok)
            return off + plsc.all_reduce_population_count(m)
        pltpu.sync_copy(ci_v, cidx_hbm.at[token])
        pltpu.sync_copy(cv_v, cval_hbm.at[token])
        return 0

    lax.fori_loop(0, TPW, tok_body, 0)


def _sc_compact(fa, tau):
    kern = pl.kernel(
        _sc_compact_body,
        out_type=[
            jax.ShapeDtypeStruct((NT, CAP), jnp.int32),
            jax.ShapeDtypeStruct((NT, CAP), jnp.float32),
        ],
        mesh=plsc.VectorSubcoreMesh(core_axis_name="c", subcore_axis_name="s"),
        compiler_params=pltpu.CompilerParams(needs_layout_passes=False),
        scratch_types=[
            pltpu.VMEM((TPW + LANES,), jnp.float32),
            pltpu.VMEM((NL,), jnp.float32),
            pltpu.VMEM((CAP,), jnp.int32),
            pltpu.VMEM((CAP,), jnp.float32),
        ],
    )
    return kern(fa, tau)


def _select_body(cv_ref, ci_ref, ta_ref, ti_ref, val_scr):
    val_scr[...] = cv_ref[...]
    idx = ci_ref[...]
    for k in range(K):
        val = val_scr[...]
        m = jnp.max(val, axis=1, keepdims=True)
        sel = jnp.min(jnp.where(val == m, idx, jnp.int32(2 ** 30)),
                      axis=1, keepdims=True)
        ta_ref[:, pl.ds(k, 1)] = m
        ti_ref[:, pl.ds(k, 1)] = sel
        val_scr[...] = jnp.where(idx == sel, -jnp.inf, val)


def _select(cval, cidx):
    return pl.pallas_call(
        _select_body,
        out_shape=[
            jax.ShapeDtypeStruct((NT, K), jnp.float32),
            jax.ShapeDtypeStruct((NT, K), jnp.int32),
        ],
        scratch_shapes=[pltpu.VMEM((NT, CAP), jnp.float32)],
    )(cval, cidx)


def _sc_decode_body(wdec_hbm, ti_hbm, ta_hbm, y_hbm,
                    idx_v, act_v, rows_v, y_v, sem):
    wid = lax.axis_index("s") * NC + lax.axis_index("c")
    base = wid * TPW

    def tok_body(t, _):
        token = base + t
        pltpu.sync_copy(ti_hbm.at[pl.ds(token * K, K)], idx_v)
        pltpu.sync_copy(ta_hbm.at[pl.ds(token * K, K)], act_v.at[pl.ds(0, K)])
        pltpu.async_copy(wdec_hbm.at[idx_v], rows_v, sem).wait()
        for chunk in range(D_IN // 256):

            @plsc.parallel_loop(
                0, K,
                carry=tuple(jnp.zeros((LANES,), jnp.float32)
                            for _ in range(16)),
                unroll=4)
            def rbody(r, acc):
                a = jnp.full((LANES,), act_v[pl.ds(r, LANES)][0])
                return tuple(
                    acc[q] + rows_v[r, pl.ds(chunk * 256 + q * LANES, LANES)] * a
                    for q in range(16))

            acc = rbody
            for q in range(16):
                y_v[pl.ds(chunk * 256 + q * LANES, LANES)] = acc[q]
        pltpu.sync_copy(y_v, y_hbm.at[pl.ds(token * D_IN, D_IN)])
        return 0

    lax.fori_loop(0, TPW, tok_body, 0)


def _sc_decode(w_dec, top_idx, top_acts):
    kern = pl.kernel(
        _sc_decode_body,
        out_type=jax.ShapeDtypeStruct((NT * D_IN,), jnp.float32),
        mesh=plsc.VectorSubcoreMesh(core_axis_name="c", subcore_axis_name="s"),
        compiler_params=pltpu.CompilerParams(needs_layout_passes=False),
        scratch_types=[
            pltpu.VMEM((K,), jnp.int32),
            pltpu.VMEM((K + LANES,), jnp.float32),
            pltpu.VMEM((K, D_IN), jnp.float32),
            pltpu.VMEM((D_IN,), jnp.float32),
            pltpu.SemaphoreType.DMA,
        ],
    )
    return kern(w_dec, top_idx.reshape(NT * K),
                top_acts.reshape(NT * K)).reshape(NT, D_IN)


def _final_body(x_ref, y_ref, bdec_ref, out_ref, fvu_ref, l2_ref):
    x = x_ref[...]
    so = y_ref[...] + bdec_ref[...]
    out_ref[...] = so
    e = so - x
    l2 = jnp.sum(e * e)
    mu = jnp.mean(x, axis=0, keepdims=True)
    tv = jnp.sum((x - mu) ** 2)
    l2_ref[...] = l2.reshape(1, 1)
    fvu_ref[...] = (l2 / tv).reshape(1, 1)


def _finalize(x, y, b_dec):
    return pl.pallas_call(
        _final_body,
        out_shape=[
            jax.ShapeDtypeStruct((NT, D_IN), jnp.float32),
            jax.ShapeDtypeStruct((1, 1), jnp.float32),
            jax.ShapeDtypeStruct((1, 1), jnp.float32),
        ],
    )(x, y, b_dec.reshape(1, D_IN))


def kernel(x, W_enc, b_enc, W_dec, b_dec):
    fa, bm = _encode(x, W_enc, b_enc, b_dec)
    tau = _tau(bm)
    cidx, cval = _sc_compact(fa, tau.reshape(NT))
    top_acts, top_idx = _select(cval, cidx)
    y = _sc_decode(W_dec, top_idx, top_acts)
    sae_out, fvu, l2 = _finalize(x, y, b_dec)
    z = jnp.zeros((), x.dtype)
    return (sae_out, fa, top_acts, top_idx, fvu.reshape(()), z, z, z,
            l2.reshape(()))


# speedup vs baseline: 2.8852x; 1.2077x over previous
import functools

import jax
import jax.numpy as jnp
from jax import lax
from jax.experimental import pallas as pl
from jax.experimental.pallas import tpu as pltpu
from jax.experimental.pallas import tpu_sc as plsc

D_IN = 1024
NL = 32768
K = 64
NT = 2048

LT = 1024
NSTEP = NL // LT
BPB = 128
NBS = LT // BPB
CAP = 128

NC, NS, LANES = 2, 16, 16
NW = NC * NS
TPW = NT // NW


TT = 1024
NTT = NT // TT


def _encode_body(x_ref, w_ref, benc_ref, bdec_ref, fa_ref, bm_ref, sae_scr):
    i = pl.program_id(0)
    j = pl.program_id(1)

    @pl.when(jnp.logical_and(i == 0, j == 0))
    def _():
        sae_scr[...] = x_ref[...] - bdec_ref[...]

    lhs = sae_scr[pl.ds(j * TT, TT), :]
    pre = lax.dot_general(lhs, w_ref[...],
                          (((1,), (1,)), ((), ())),
                          preferred_element_type=jnp.float32)
    fa = jnp.maximum(pre + benc_ref[0], 0.0)
    fa_ref[...] = fa
    bm_ref[0] = jnp.max(fa.reshape(TT, NBS, BPB), axis=2).T


def _encode(x, w_enc, b_enc, b_dec):
    return pl.pallas_call(
        _encode_body,
        grid=(NSTEP, NTT),
        in_specs=[
            pl.BlockSpec((NT, D_IN), lambda i, j: (0, 0)),
            pl.BlockSpec((LT, D_IN), lambda i, j: (i, 0)),
            pl.BlockSpec((1, 1, LT), lambda i, j: (i, 0, 0)),
            pl.BlockSpec((1, D_IN), lambda i, j: (0, 0)),
        ],
        out_specs=[
            pl.BlockSpec((TT, LT), lambda i, j: (j, i)),
            pl.BlockSpec((1, NBS, TT), lambda i, j: (i, 0, j)),
        ],
        out_shape=[
            jax.ShapeDtypeStruct((NT, NL), jnp.float32),
            jax.ShapeDtypeStruct((NSTEP, NBS, NT), jnp.float32),
        ],
        scratch_shapes=[
            pltpu.VMEM((NT, D_IN), jnp.float32),
        ],
        compiler_params=pltpu.CompilerParams(
            vmem_limit_bytes=36 * 1024 * 1024),
    )(x, w_enc, b_enc.reshape(NSTEP, 1, LT), b_dec.reshape(1, D_IN))


def _tau_body(bm_ref, tau_ref):
    bits = lax.bitcast_convert_type(
        bm_ref[...].reshape(NSTEP * NBS, NT), jnp.int32)

    def search(it, t):
        cand = t | (jnp.int32(1) << (30 - it))
        cnt = jnp.sum((bits >= cand).astype(jnp.int32), axis=0,
                      keepdims=True)
        return jnp.where(cnt >= K, cand, t)

    t = lax.fori_loop(0, 31, search, jnp.zeros((1, NT), jnp.int32))
    tau_ref[...] = lax.bitcast_convert_type(t, jnp.float32)


def _tau(bm):
    return pl.pallas_call(
        _tau_body,
        out_shape=jax.ShapeDtypeStruct((1, NT), jnp.float32),
    )(bm)


def _sc_compact_body(fa_hbm, tau_hbm, cidx_hbm, cval_hbm,
                     tau_v, row_v, ci_v, cv_v):
    wid = lax.axis_index("s") * NC + lax.axis_index("c")
    base = wid * TPW
    pltpu.sync_copy(tau_hbm.at[pl.ds(base, TPW)], tau_v.at[pl.ds(0, TPW)])

    def tok_body(ti, _):
        token = base + ti
        pltpu.sync_copy(fa_hbm.at[token], row_v)
        for q in range(CAP // LANES):
            ci_v[pl.ds(q * LANES, LANES)] = jnp.zeros((LANES,), jnp.int32)
            cv_v[pl.ds(q * LANES, LANES)] = jnp.full((LANES,), -1.0,
                                                     jnp.float32)
        tau_vec = jnp.full((LANES,), tau_v[pl.ds(ti, LANES)][0])

        @plsc.parallel_loop(0, NL // LANES,
                            carry=jnp.zeros((LANES,), jnp.int32), unroll=8)
        def vbody(j, off):
            v = row_v[pl.ds(j * LANES, LANES)]
            m = v >= tau_vec
            scan = plsc.cumsum(m.astype(jnp.int32))
            pos = off + scan - 1
            ok = jnp.logical_and(m, pos < CAP)
            idxv = lax.iota(jnp.int32, LANES) + j * LANES
            plsc.store_scatter(ci_v, [pos], idxv, mask=ok)
            plsc.store_scatter(cv_v, [pos], v, mask=ok)
            return off + plsc.all_reduce_population_count(m)
        pltpu.sync_copy(ci_v, cidx_hbm.at[token])
        pltpu.sync_copy(cv_v, cval_hbm.at[token])
        return 0

    lax.fori_loop(0, TPW, tok_body, 0)


def _sc_compact(fa, tau):
    kern = pl.kernel(
        _sc_compact_body,
        out_type=[
            jax.ShapeDtypeStruct((NT, CAP), jnp.int32),
            jax.ShapeDtypeStruct((NT, CAP), jnp.float32),
        ],
        mesh=plsc.VectorSubcoreMesh(core_axis_name="c", subcore_axis_name="s"),
        compiler_params=pltpu.CompilerParams(needs_layout_passes=False),
        scratch_types=[
            pltpu.VMEM((TPW + LANES,), jnp.float32),
            pltpu.VMEM((NL,), jnp.float32),
            pltpu.VMEM((CAP,), jnp.int32),
            pltpu.VMEM((CAP,), jnp.float32),
        ],
    )
    return kern(fa, tau)


def _select_body(cv_ref, ci_ref, ta_ref, ti_ref, val_scr):
    val_scr[...] = cv_ref[...]
    idx = ci_ref[...]
    for k in range(K):
        val = val_scr[...]
        m = jnp.max(val, axis=1, keepdims=True)
        sel = jnp.min(jnp.where(val == m, idx, jnp.int32(2 ** 30)),
                      axis=1, keepdims=True)
        ta_ref[:, pl.ds(k, 1)] = m
        ti_ref[:, pl.ds(k, 1)] = sel
        val_scr[...] = jnp.where(idx == sel, -jnp.inf, val)


def _select(cval, cidx):
    return pl.pallas_call(
        _select_body,
        out_shape=[
            jax.ShapeDtypeStruct((NT, K), jnp.float32),
            jax.ShapeDtypeStruct((NT, K), jnp.int32),
        ],
        scratch_shapes=[pltpu.VMEM((NT, CAP), jnp.float32)],
    )(cval, cidx)


HALF = K // 2


def _sc_decode_body(wdec_hbm, ti_hbm, ta_hbm, y_hbm,
                    idxs_v, acts_v, rows0_v, rows1_v, y_v, sem0, sem1):
    wid = lax.axis_index("s") * NC + lax.axis_index("c")
    base = wid * TPW
    pltpu.sync_copy(ti_hbm.at[pl.ds(base * K, TPW * K)], idxs_v)
    pltpu.sync_copy(ta_hbm.at[pl.ds(base * K, TPW * K)],
                    acts_v.at[pl.ds(0, TPW * K)])

    def start(h, buf, sem):
        hc = jnp.minimum(h, 2 * TPW - 1)
        pltpu.make_async_copy(
            wdec_hbm.at[idxs_v.at[pl.ds(hc * HALF, HALF)]], buf, sem).start()

    def wait(buf, sem):
        pltpu.make_async_copy(
            wdec_hbm.at[idxs_v.at[pl.ds(0, HALF)]], buf, sem).wait()

    def accumulate(buf, abase, first):
        for chunk in range(D_IN // 256):

            @plsc.parallel_loop(
                0, HALF,
                carry=tuple(jnp.zeros((LANES,), jnp.float32)
                            for _ in range(16)),
                unroll=4)
            def rbody(r, acc):
                a = jnp.full((LANES,), acts_v[pl.ds(abase + r, LANES)][0])
                return tuple(
                    acc[q] + buf[r, pl.ds(chunk * 256 + q * LANES, LANES)] * a
                    for q in range(16))

            acc = rbody
            for q in range(16):
                sl = pl.ds(chunk * 256 + q * LANES, LANES)
                if first:
                    y_v[sl] = acc[q]
                else:
                    plsc.addupdate(y_v.at[sl], acc[q])

    start(0, rows0_v, sem0)
    start(1, rows1_v, sem1)

    def tok_body(t, _):
        token = base + t
        wait(rows0_v, sem0)
        accumulate(rows0_v, t * K, True)
        start(2 * (t + 1), rows0_v, sem0)
        wait(rows1_v, sem1)
        accumulate(rows1_v, t * K + HALF, False)
        pltpu.sync_copy(y_v, y_hbm.at[pl.ds(token * D_IN, D_IN)])
        start(2 * (t + 1) + 1, rows1_v, sem1)
        return 0

    lax.fori_loop(0, TPW, tok_body, 0)
    wait(rows0_v, sem0)
    wait(rows1_v, sem1)


def _sc_decode(w_dec, top_idx, top_acts):
    kern = pl.kernel(
        _sc_decode_body,
        out_type=jax.ShapeDtypeStruct((NT * D_IN,), jnp.float32),
        mesh=plsc.VectorSubcoreMesh(core_axis_name="c", subcore_axis_name="s"),
        compiler_params=pltpu.CompilerParams(needs_layout_passes=False),
        scratch_types=[
            pltpu.VMEM((TPW * K,), jnp.int32),
            pltpu.VMEM((TPW * K + LANES,), jnp.float32),
            pltpu.VMEM((HALF, D_IN), jnp.float32),
            pltpu.VMEM((HALF, D_IN), jnp.float32),
            pltpu.VMEM((D_IN,), jnp.float32),
            pltpu.SemaphoreType.DMA,
            pltpu.SemaphoreType.DMA,
        ],
    )
    return kern(w_dec, top_idx.reshape(NT * K),
                top_acts.reshape(NT * K)).reshape(NT, D_IN)


def _final_body(x_ref, y_ref, bdec_ref, out_ref, fvu_ref, l2_ref):
    x = x_ref[...]
    so = y_ref[...] + bdec_ref[...]
    out_ref[...] = so
    e = so - x
    l2 = jnp.sum(e * e)
    mu = jnp.mean(x, axis=0, keepdims=True)
    tv = jnp.sum((x - mu) ** 2)
    l2_ref[...] = l2.reshape(1, 1)
    fvu_ref[...] = (l2 / tv).reshape(1, 1)


def _finalize(x, y, b_dec):
    return pl.pallas_call(
        _final_body,
        out_shape=[
            jax.ShapeDtypeStruct((NT, D_IN), jnp.float32),
            jax.ShapeDtypeStruct((1, 1), jnp.float32),
            jax.ShapeDtypeStruct((1, 1), jnp.float32),
        ],
    )(x, y, b_dec.reshape(1, D_IN))


def kernel(x, W_enc, b_enc, W_dec, b_dec):
    fa, bm = _encode(x, W_enc, b_enc, b_dec)
    tau = _tau(bm)
    cidx, cval = _sc_compact(fa, tau.reshape(NT))
    top_acts, top_idx = _select(cval, cidx)
    y = _sc_decode(W_dec, top_idx, top_acts)
    sae_out, fvu, l2 = _finalize(x, y, b_dec)
    z = jnp.zeros((), x.dtype)
    return (sae_out, fa, top_acts, top_idx, fvu.reshape(()), z, z, z,
            l2.reshape(()))


# speedup vs baseline: 3.3150x; 1.1490x over previous
import functools

import jax
import jax.numpy as jnp
from jax import lax
from jax.experimental import pallas as pl
from jax.experimental.pallas import tpu as pltpu
from jax.experimental.pallas import tpu_sc as plsc

D_IN = 1024
NL = 32768
K = 64
NT = 2048

LT = 1024
NSTEP = NL // LT
BPB = 128
NBS = LT // BPB
CAP = 128

NC, NS, LANES = 2, 16, 16
NW = NC * NS
TPW = NT // NW


TT = 1024
NTT = NT // TT


def _encode_body(x_ref, w_ref, benc_ref, bdec_ref, fa_ref, bm_ref, sae_scr):
    i = pl.program_id(0)
    j = pl.program_id(1)

    @pl.when(jnp.logical_and(i == 0, j == 0))
    def _():
        sae_scr[...] = x_ref[...] - bdec_ref[...]

    lhs = sae_scr[pl.ds(j * TT, TT), :]
    pre = lax.dot_general(lhs, w_ref[...],
                          (((1,), (1,)), ((), ())),
                          preferred_element_type=jnp.float32)
    fa = jnp.maximum(pre + benc_ref[0], 0.0)
    fa_ref[...] = fa
    bm_ref[0] = jnp.max(fa.reshape(TT, NBS, BPB), axis=2).T


def _encode(x, w_enc, b_enc, b_dec):
    return pl.pallas_call(
        _encode_body,
        grid=(NSTEP, NTT),
        in_specs=[
            pl.BlockSpec((NT, D_IN), lambda i, j: (0, 0)),
            pl.BlockSpec((LT, D_IN), lambda i, j: (i, 0)),
            pl.BlockSpec((1, 1, LT), lambda i, j: (i, 0, 0)),
            pl.BlockSpec((1, D_IN), lambda i, j: (0, 0)),
        ],
        out_specs=[
            pl.BlockSpec((TT, LT), lambda i, j: (j, i)),
            pl.BlockSpec((1, NBS, TT), lambda i, j: (i, 0, j)),
        ],
        out_shape=[
            jax.ShapeDtypeStruct((NT, NL), jnp.float32),
            jax.ShapeDtypeStruct((NSTEP, NBS, NT), jnp.float32),
        ],
        scratch_shapes=[
            pltpu.VMEM((NT, D_IN), jnp.float32),
        ],
        compiler_params=pltpu.CompilerParams(
            vmem_limit_bytes=36 * 1024 * 1024),
    )(x, w_enc, b_enc.reshape(NSTEP, 1, LT), b_dec.reshape(1, D_IN))


def _tau_body(bm_ref, tau_ref):
    bits = lax.bitcast_convert_type(
        bm_ref[...].reshape(NSTEP * NBS, NT), jnp.int32)

    def search(it, t):
        cand = t | (jnp.int32(1) << (30 - it))
        cnt = jnp.sum((bits >= cand).astype(jnp.int32), axis=0,
                      keepdims=True)
        return jnp.where(cnt >= K, cand, t)

    t = lax.fori_loop(0, 31, search, jnp.zeros((1, NT), jnp.int32))
    tau_ref[...] = lax.bitcast_convert_type(t, jnp.float32)


def _tau(bm):
    return pl.pallas_call(
        _tau_body,
        out_shape=jax.ShapeDtypeStruct((1, NT), jnp.float32),
    )(bm)


def _sc_compact_body(fa_hbm, tau_hbm, cidx_hbm, cval_hbm, tau_v,
                     row0_v, row1_v, ci0_v, cv0_v, ci1_v, cv1_v,
                     semr0, semr1, semo0, semo1):
    wid = lax.axis_index("s") * NC + lax.axis_index("c")
    base = wid * TPW
    pltpu.sync_copy(tau_hbm.at[pl.ds(base, TPW)], tau_v.at[pl.ds(0, TPW)])

    def start_row(ti, buf, sem):
        tc = jnp.minimum(ti, TPW - 1)
        pltpu.make_async_copy(fa_hbm.at[base + tc], buf, sem).start()

    def wait_row(buf, sem):
        pltpu.make_async_copy(fa_hbm.at[base], buf, sem).wait()

    def process(ti, row_v, ci_v, cv_v, semo, t):
        @pl.when(t > 0)
        def _():
            pltpu.make_async_copy(ci_v, cidx_hbm.at[base], semo).wait()
            pltpu.make_async_copy(cv_v, cval_hbm.at[base], semo).wait()

        for q in range(CAP // LANES):
            ci_v[pl.ds(q * LANES, LANES)] = jnp.zeros((LANES,), jnp.int32)
            cv_v[pl.ds(q * LANES, LANES)] = jnp.full((LANES,), -1.0,
                                                     jnp.float32)
        tau_vec = jnp.full((LANES,), tau_v[pl.ds(ti, LANES)][0])

        @plsc.parallel_loop(0, NL // LANES,
                            carry=jnp.zeros((LANES,), jnp.int32), unroll=8)
        def vbody(j, off):
            v = row_v[pl.ds(j * LANES, LANES)]
            m = v >= tau_vec
            scan = plsc.cumsum(m.astype(jnp.int32))
            pos = off + scan - 1
            ok = jnp.logical_and(m, pos < CAP)
            idxv = lax.iota(jnp.int32, LANES) + j * LANES
            plsc.store_scatter(ci_v, [pos], idxv, mask=ok)
            plsc.store_scatter(cv_v, [pos], v, mask=ok)
            return off + plsc.all_reduce_population_count(m)

        pltpu.make_async_copy(ci_v, cidx_hbm.at[base + ti], semo).start()
        pltpu.make_async_copy(cv_v, cval_hbm.at[base + ti], semo).start()

    start_row(0, row0_v, semr0)
    start_row(1, row1_v, semr1)

    def tok_body(t, _):
        wait_row(row0_v, semr0)
        process(2 * t, row0_v, ci0_v, cv0_v, semo0, t)
        start_row(2 * (t + 1), row0_v, semr0)
        wait_row(row1_v, semr1)
        process(2 * t + 1, row1_v, ci1_v, cv1_v, semo1, t)
        start_row(2 * (t + 1) + 1, row1_v, semr1)
        return 0

    lax.fori_loop(0, TPW // 2, tok_body, 0)
    wait_row(row0_v, semr0)
    wait_row(row1_v, semr1)
    pltpu.make_async_copy(ci0_v, cidx_hbm.at[base], semo0).wait()
    pltpu.make_async_copy(cv0_v, cval_hbm.at[base], semo0).wait()
    pltpu.make_async_copy(ci1_v, cidx_hbm.at[base], semo1).wait()
    pltpu.make_async_copy(cv1_v, cval_hbm.at[base], semo1).wait()


def _sc_compact(fa, tau):
    kern = pl.kernel(
        _sc_compact_body,
        out_type=[
            jax.ShapeDtypeStruct((NT, CAP), jnp.int32),
            jax.ShapeDtypeStruct((NT, CAP), jnp.float32),
        ],
        mesh=plsc.VectorSubcoreMesh(core_axis_name="c", subcore_axis_name="s"),
        compiler_params=pltpu.CompilerParams(needs_layout_passes=False),
        scratch_types=[
            pltpu.VMEM((TPW + LANES,), jnp.float32),
            pltpu.VMEM((NL,), jnp.float32),
            pltpu.VMEM((NL,), jnp.float32),
            pltpu.VMEM((CAP,), jnp.int32),
            pltpu.VMEM((CAP,), jnp.float32),
            pltpu.VMEM((CAP,), jnp.int32),
            pltpu.VMEM((CAP,), jnp.float32),
            pltpu.SemaphoreType.DMA,
            pltpu.SemaphoreType.DMA,
            pltpu.SemaphoreType.DMA,
            pltpu.SemaphoreType.DMA,
        ],
    )
    return kern(fa, tau)


def _select_body(cv_ref, ci_ref, ta_ref, ti_ref, val_scr):
    val_scr[...] = cv_ref[...]
    idx = ci_ref[...]
    for k in range(K):
        val = val_scr[...]
        m = jnp.max(val, axis=1, keepdims=True)
        sel = jnp.min(jnp.where(val == m, idx, jnp.int32(2 ** 30)),
                      axis=1, keepdims=True)
        ta_ref[:, pl.ds(k, 1)] = m
        ti_ref[:, pl.ds(k, 1)] = sel
        val_scr[...] = jnp.where(idx == sel, -jnp.inf, val)


def _select(cval, cidx):
    return pl.pallas_call(
        _select_body,
        out_shape=[
            jax.ShapeDtypeStruct((NT, K), jnp.float32),
            jax.ShapeDtypeStruct((NT, K), jnp.int32),
        ],
        scratch_shapes=[pltpu.VMEM((NT, CAP), jnp.float32)],
    )(cval, cidx)


HALF = K // 2


def _sc_decode_body(wdec_hbm, ti_hbm, ta_hbm, y_hbm,
                    idxs_v, acts_v, rows0_v, rows1_v, y_v, sem0, sem1):
    wid = lax.axis_index("s") * NC + lax.axis_index("c")
    base = wid * TPW
    pltpu.sync_copy(ti_hbm.at[pl.ds(base * K, TPW * K)], idxs_v)
    pltpu.sync_copy(ta_hbm.at[pl.ds(base * K, TPW * K)],
                    acts_v.at[pl.ds(0, TPW * K)])

    def start(h, buf, sem):
        hc = jnp.minimum(h, 2 * TPW - 1)
        pltpu.make_async_copy(
            wdec_hbm.at[idxs_v.at[pl.ds(hc * HALF, HALF)]], buf, sem).start()

    def wait(buf, sem):
        pltpu.make_async_copy(
            wdec_hbm.at[idxs_v.at[pl.ds(0, HALF)]], buf, sem).wait()

    def accumulate(buf, abase, first):
        for chunk in range(D_IN // 256):

            @plsc.parallel_loop(
                0, HALF,
                carry=tuple(jnp.zeros((LANES,), jnp.float32)
                            for _ in range(16)),
                unroll=4)
            def rbody(r, acc):
                a = jnp.full((LANES,), acts_v[pl.ds(abase + r, LANES)][0])
                return tuple(
                    acc[q] + buf[r, pl.ds(chunk * 256 + q * LANES, LANES)] * a
                    for q in range(16))

            acc = rbody
            for q in range(16):
                sl = pl.ds(chunk * 256 + q * LANES, LANES)
                if first:
                    y_v[sl] = acc[q]
                else:
                    plsc.addupdate(y_v.at[sl], acc[q])

    start(0, rows0_v, sem0)
    start(1, rows1_v, sem1)

    def tok_body(t, _):
        token = base + t
        wait(rows0_v, sem0)
        accumulate(rows0_v, t * K, True)
        start(2 * (t + 1), rows0_v, sem0)
        wait(rows1_v, sem1)
        accumulate(rows1_v, t * K + HALF, False)
        pltpu.sync_copy(y_v, y_hbm.at[pl.ds(token * D_IN, D_IN)])
        start(2 * (t + 1) + 1, rows1_v, sem1)
        return 0

    lax.fori_loop(0, TPW, tok_body, 0)
    wait(rows0_v, sem0)
    wait(rows1_v, sem1)


def _sc_decode(w_dec, top_idx, top_acts):
    kern = pl.kernel(
        _sc_decode_body,
        out_type=jax.ShapeDtypeStruct((NT * D_IN,), jnp.float32),
        mesh=plsc.VectorSubcoreMesh(core_axis_name="c", subcore_axis_name="s"),
        compiler_params=pltpu.CompilerParams(needs_layout_passes=False),
        scratch_types=[
            pltpu.VMEM((TPW * K,), jnp.int32),
            pltpu.VMEM((TPW * K + LANES,), jnp.float32),
            pltpu.VMEM((HALF, D_IN), jnp.float32),
            pltpu.VMEM((HALF, D_IN), jnp.float32),
            pltpu.VMEM((D_IN,), jnp.float32),
            pltpu.SemaphoreType.DMA,
            pltpu.SemaphoreType.DMA,
        ],
    )
    return kern(w_dec, top_idx.reshape(NT * K),
                top_acts.reshape(NT * K)).reshape(NT, D_IN)


def _final_body(x_ref, y_ref, bdec_ref, out_ref, fvu_ref, l2_ref):
    x = x_ref[...]
    so = y_ref[...] + bdec_ref[...]
    out_ref[...] = so
    e = so - x
    l2 = jnp.sum(e * e)
    mu = jnp.mean(x, axis=0, keepdims=True)
    tv = jnp.sum((x - mu) ** 2)
    l2_ref[...] = l2.reshape(1, 1)
    fvu_ref[...] = (l2 / tv).reshape(1, 1)


def _finalize(x, y, b_dec):
    return pl.pallas_call(
        _final_body,
        out_shape=[
            jax.ShapeDtypeStruct((NT, D_IN), jnp.float32),
            jax.ShapeDtypeStruct((1, 1), jnp.float32),
            jax.ShapeDtypeStruct((1, 1), jnp.float32),
        ],
    )(x, y, b_dec.reshape(1, D_IN))


def kernel(x, W_enc, b_enc, W_dec, b_dec):
    fa, bm = _encode(x, W_enc, b_enc, b_dec)
    tau = _tau(bm)
    cidx, cval = _sc_compact(fa, tau.reshape(NT))
    top_acts, top_idx = _select(cval, cidx)
    y = _sc_decode(W_dec, top_idx, top_acts)
    sae_out, fvu, l2 = _finalize(x, y, b_dec)
    z = jnp.zeros((), x.dtype)
    return (sae_out, fa, top_acts, top_idx, fvu.reshape(()), z, z, z,
            l2.reshape(()))


# speedup vs baseline: 3.3157x; 1.0002x over previous
import jax
import jax.numpy as jnp
from jax import lax
from jax.experimental import pallas as pl
from jax.experimental.pallas import tpu as pltpu
from jax.experimental.pallas import tpu_sc as plsc

D_IN = 1024
NL = 32768
K = 64
NT = 2048

LT = 1024
NSTEP = NL // LT
BPB = 128
NBS = LT // BPB
CAP = 128

NC, NS, LANES = 2, 16, 16
NW = NC * NS
TPW = NT // NW


TT = 1024
NTT = NT // TT


def _encode_body(x_ref, w_ref, benc_ref, bdec_ref, fa_ref, bm_ref, sae_scr):
    i = pl.program_id(0)
    j = pl.program_id(1)

    @pl.when(jnp.logical_and(i == 0, j == 0))
    def _():
        sae_scr[...] = x_ref[...] - bdec_ref[...]

    lhs = sae_scr[pl.ds(j * TT, TT), :]
    pre = lax.dot_general(lhs, w_ref[...],
                          (((1,), (1,)), ((), ())),
                          preferred_element_type=jnp.float32)
    fa = jnp.maximum(pre + benc_ref[0], 0.0)
    fa_ref[...] = fa
    bm_ref[0] = jnp.max(fa.reshape(TT, NBS, BPB), axis=2).T


def _encode(x, w_enc, b_enc, b_dec):
    return pl.pallas_call(
        _encode_body,
        grid=(NSTEP, NTT),
        in_specs=[
            pl.BlockSpec((NT, D_IN), lambda i, j: (0, 0)),
            pl.BlockSpec((LT, D_IN), lambda i, j: (i, 0)),
            pl.BlockSpec((1, 1, LT), lambda i, j: (i, 0, 0)),
            pl.BlockSpec((1, D_IN), lambda i, j: (0, 0)),
        ],
        out_specs=[
            pl.BlockSpec((TT, LT), lambda i, j: (j, i)),
            pl.BlockSpec((1, NBS, TT), lambda i, j: (i, 0, j)),
        ],
        out_shape=[
            jax.ShapeDtypeStruct((NT, NL), jnp.float32),
            jax.ShapeDtypeStruct((NSTEP, NBS, NT), jnp.float32),
        ],
        scratch_shapes=[
            pltpu.VMEM((NT, D_IN), jnp.float32),
        ],
        compiler_params=pltpu.CompilerParams(
            vmem_limit_bytes=36 * 1024 * 1024),
    )(x, w_enc, b_enc.reshape(NSTEP, 1, LT), b_dec.reshape(1, D_IN))


def _tau_body(bm_ref, tau_ref):
    bits = lax.bitcast_convert_type(
        bm_ref[...].reshape(NSTEP * NBS, NT), jnp.int32)

    def search(it, t):
        cand = t | (jnp.int32(1) << (30 - it))
        cnt = jnp.sum((bits >= cand).astype(jnp.int32), axis=0,
                      keepdims=True)
        return jnp.where(cnt >= K, cand, t)

    t = lax.fori_loop(0, 31, search, jnp.zeros((1, NT), jnp.int32))
    tau_ref[...] = lax.bitcast_convert_type(t, jnp.float32)


def _tau(bm):
    return pl.pallas_call(
        _tau_body,
        out_shape=jax.ShapeDtypeStruct((1, NT), jnp.float32),
    )(bm)


def _sc_compact_body(fa_hbm, tau_hbm, cidx_hbm, cval_hbm, tau_v,
                     row0_v, row1_v, ci0_v, cv0_v, ci1_v, cv1_v,
                     semr0, semr1, semo0, semo1):
    wid = lax.axis_index("s") * NC + lax.axis_index("c")
    base = wid * TPW
    pltpu.sync_copy(tau_hbm.at[pl.ds(base, TPW)], tau_v.at[pl.ds(0, TPW)])

    def start_row(ti, buf, sem):
        tc = jnp.minimum(ti, TPW - 1)
        pltpu.make_async_copy(fa_hbm.at[base + tc], buf, sem).start()

    def wait_row(buf, sem):
        pltpu.make_async_copy(fa_hbm.at[base], buf, sem).wait()

    def process(ti, row_v, ci_v, cv_v, semo, t):
        @pl.when(t > 0)
        def _():
            pltpu.make_async_copy(ci_v, cidx_hbm.at[base], semo).wait()
            pltpu.make_async_copy(cv_v, cval_hbm.at[base], semo).wait()

        for q in range(CAP // LANES):
            ci_v[pl.ds(q * LANES, LANES)] = jnp.zeros((LANES,), jnp.int32)
            cv_v[pl.ds(q * LANES, LANES)] = jnp.full((LANES,), -1.0,
                                                     jnp.float32)
        tau_vec = jnp.full((LANES,), tau_v[pl.ds(ti, LANES)][0])

        @plsc.parallel_loop(0, NL // LANES,
                            carry=jnp.zeros((LANES,), jnp.int32), unroll=8)
        def vbody(j, off):
            v = row_v[pl.ds(j * LANES, LANES)]
            m = v >= tau_vec
            scan = plsc.cumsum(m.astype(jnp.int32))
            pos = off + scan - 1
            ok = jnp.logical_and(m, pos < CAP)
            idxv = lax.iota(jnp.int32, LANES) + j * LANES
            plsc.store_scatter(ci_v, [pos], idxv, mask=ok)
            plsc.store_scatter(cv_v, [pos], v, mask=ok)
            return off + plsc.all_reduce_population_count(m)

        pltpu.make_async_copy(ci_v, cidx_hbm.at[base + ti], semo).start()
        pltpu.make_async_copy(cv_v, cval_hbm.at[base + ti], semo).start()

    start_row(0, row0_v, semr0)
    start_row(1, row1_v, semr1)

    def tok_body(t, _):
        wait_row(row0_v, semr0)
        process(2 * t, row0_v, ci0_v, cv0_v, semo0, t)
        start_row(2 * (t + 1), row0_v, semr0)
        wait_row(row1_v, semr1)
        process(2 * t + 1, row1_v, ci1_v, cv1_v, semo1, t)
        start_row(2 * (t + 1) + 1, row1_v, semr1)
        return 0

    lax.fori_loop(0, TPW // 2, tok_body, 0)
    wait_row(row0_v, semr0)
    wait_row(row1_v, semr1)
    pltpu.make_async_copy(ci0_v, cidx_hbm.at[base], semo0).wait()
    pltpu.make_async_copy(cv0_v, cval_hbm.at[base], semo0).wait()
    pltpu.make_async_copy(ci1_v, cidx_hbm.at[base], semo1).wait()
    pltpu.make_async_copy(cv1_v, cval_hbm.at[base], semo1).wait()


def _sc_compact(fa, tau):
    kern = pl.kernel(
        _sc_compact_body,
        out_type=[
            jax.ShapeDtypeStruct((NT, CAP), jnp.int32),
            jax.ShapeDtypeStruct((NT, CAP), jnp.float32),
        ],
        mesh=plsc.VectorSubcoreMesh(core_axis_name="c", subcore_axis_name="s"),
        compiler_params=pltpu.CompilerParams(needs_layout_passes=False),
        scratch_types=[
            pltpu.VMEM((TPW + LANES,), jnp.float32),
            pltpu.VMEM((NL,), jnp.float32),
            pltpu.VMEM((NL,), jnp.float32),
            pltpu.VMEM((CAP,), jnp.int32),
            pltpu.VMEM((CAP,), jnp.float32),
            pltpu.VMEM((CAP,), jnp.int32),
            pltpu.VMEM((CAP,), jnp.float32),
            pltpu.SemaphoreType.DMA,
            pltpu.SemaphoreType.DMA,
            pltpu.SemaphoreType.DMA,
            pltpu.SemaphoreType.DMA,
        ],
    )
    return kern(fa, tau)


def _select_body(cv_ref, ci_ref, ta_ref, ti_ref, val_scr):
    val_scr[...] = cv_ref[...]
    idx = ci_ref[...]
    for k in range(K):
        val = val_scr[...]
        m = jnp.max(val, axis=1, keepdims=True)
        sel = jnp.min(jnp.where(val == m, idx, jnp.int32(2 ** 30)),
                      axis=1, keepdims=True)
        ta_ref[:, pl.ds(k, 1)] = m
        ti_ref[:, pl.ds(k, 1)] = sel
        val_scr[...] = jnp.where(idx == sel, -jnp.inf, val)


def _select(cval, cidx):
    return pl.pallas_call(
        _select_body,
        out_shape=[
            jax.ShapeDtypeStruct((NT, K), jnp.float32),
            jax.ShapeDtypeStruct((NT, K), jnp.int32),
        ],
        scratch_shapes=[pltpu.VMEM((NT, CAP), jnp.float32)],
    )(cval, cidx)


HALF = K // 2


def _sc_decode_body(wdec_hbm, ti_hbm, ta_hbm, y_hbm,
                    idxs_v, acts_v, rows0_v, rows1_v, y_v, sem0, sem1):
    wid = lax.axis_index("s") * NC + lax.axis_index("c")
    base = wid * TPW
    pltpu.sync_copy(ti_hbm.at[pl.ds(base * K, TPW * K)], idxs_v)
    pltpu.sync_copy(ta_hbm.at[pl.ds(base * K, TPW * K)],
                    acts_v.at[pl.ds(0, TPW * K)])

    def start(h, buf, sem):
        hc = jnp.minimum(h, 2 * TPW - 1)
        pltpu.make_async_copy(
            wdec_hbm.at[idxs_v.at[pl.ds(hc * HALF, HALF)]], buf, sem).start()

    def wait(buf, sem):
        pltpu.make_async_copy(
            wdec_hbm.at[idxs_v.at[pl.ds(0, HALF)]], buf, sem).wait()

    def accumulate(buf, abase, first):
        for chunk in range(D_IN // 256):

            @plsc.parallel_loop(
                0, HALF,
                carry=tuple(jnp.zeros((LANES,), jnp.float32)
                            for _ in range(16)),
                unroll=4)
            def rbody(r, acc):
                a = jnp.full((LANES,), acts_v[pl.ds(abase + r, LANES)][0])
                return tuple(
                    acc[q] + buf[r, pl.ds(chunk * 256 + q * LANES, LANES)] * a
                    for q in range(16))

            acc = rbody
            for q in range(16):
                sl = pl.ds(chunk * 256 + q * LANES, LANES)
                if first:
                    y_v[sl] = acc[q]
                else:
                    plsc.addupdate(y_v.at[sl], acc[q])

    start(0, rows0_v, sem0)
    start(1, rows1_v, sem1)

    def tok_body(t, _):
        token = base + t
        wait(rows0_v, sem0)
        accumulate(rows0_v, t * K, True)
        start(2 * (t + 1), rows0_v, sem0)
        wait(rows1_v, sem1)
        accumulate(rows1_v, t * K + HALF, False)
        pltpu.sync_copy(y_v, y_hbm.at[pl.ds(token * D_IN, D_IN)])
        start(2 * (t + 1) + 1, rows1_v, sem1)
        return 0

    lax.fori_loop(0, TPW, tok_body, 0)
    wait(rows0_v, sem0)
    wait(rows1_v, sem1)


def _sc_decode(w_dec, top_idx, top_acts):
    kern = pl.kernel(
        _sc_decode_body,
        out_type=jax.ShapeDtypeStruct((NT * D_IN,), jnp.float32),
        mesh=plsc.VectorSubcoreMesh(core_axis_name="c", subcore_axis_name="s"),
        compiler_params=pltpu.CompilerParams(needs_layout_passes=False),
        scratch_types=[
            pltpu.VMEM((TPW * K,), jnp.int32),
            pltpu.VMEM((TPW * K + LANES,), jnp.float32),
            pltpu.VMEM((HALF, D_IN), jnp.float32),
            pltpu.VMEM((HALF, D_IN), jnp.float32),
            pltpu.VMEM((D_IN,), jnp.float32),
            pltpu.SemaphoreType.DMA,
            pltpu.SemaphoreType.DMA,
        ],
    )
    return kern(w_dec, top_idx.reshape(NT * K),
                top_acts.reshape(NT * K)).reshape(NT, D_IN)


def _final_body(x_ref, y_ref, bdec_ref, out_ref, fvu_ref, l2_ref):
    x = x_ref[...]
    so = y_ref[...] + bdec_ref[...]
    out_ref[...] = so
    e = so - x
    l2 = jnp.sum(e * e)
    mu = jnp.mean(x, axis=0, keepdims=True)
    tv = jnp.sum((x - mu) ** 2)
    l2_ref[...] = l2.reshape(1, 1)
    fvu_ref[...] = (l2 / tv).reshape(1, 1)


def _finalize(x, y, b_dec):
    return pl.pallas_call(
        _final_body,
        out_shape=[
            jax.ShapeDtypeStruct((NT, D_IN), jnp.float32),
            jax.ShapeDtypeStruct((1, 1), jnp.float32),
            jax.ShapeDtypeStruct((1, 1), jnp.float32),
        ],
    )(x, y, b_dec.reshape(1, D_IN))


def kernel(x, W_enc, b_enc, W_dec, b_dec):
    fa, bm = _encode(x, W_enc, b_enc, b_dec)
    tau = _tau(bm)
    cidx, cval = _sc_compact(fa, tau.reshape(NT))
    top_acts, top_idx = _select(cval, cidx)
    y = _sc_decode(W_dec, top_idx, top_acts)
    sae_out, fvu, l2 = _finalize(x, y, b_dec)
    z = jnp.zeros((), x.dtype)
    return (sae_out, fa, top_acts, top_idx, fvu.reshape(()), z, z, z,
            l2.reshape(()))
